# baseline (speedup 1.0000x reference)
import functools

import jax
import jax.numpy as jnp
from jax.experimental import pallas as pl
from jax.experimental.pallas import tpu as pltpu

N = 10000
E = 160000
ES = 320000
SLOPE = 0.1


def _lrelu(x):
    return jnp.where(x >= 0, x, SLOPE * x)


def _mm_kernel(x_ref, w_ref, o_ref, *, act, slope):
    acc = jnp.dot(x_ref[...], w_ref[...], preferred_element_type=jnp.float32)
    if act:
        acc = jnp.where(acc >= 0, acc, slope * acc)
    o_ref[...] = acc


def _mm_bias_kernel(x_ref, w_ref, b_ref, o_ref, *, act, slope):
    acc = jnp.dot(x_ref[...], w_ref[...], preferred_element_type=jnp.float32)
    acc = acc + b_ref[...][None, :]
    if act:
        acc = jnp.where(acc >= 0, acc, slope * acc)
    o_ref[...] = acc


def _pick_bm(m):
    for bm in (2000, 1250, 1000, 500, 250):
        if m % bm == 0:
            return bm
    return m


def _mm(x, w, b=None, act=True):
    m, k = x.shape
    n = w.shape[1]
    bm = _pick_bm(m)
    grid = (m // bm,)
    in_specs = [
        pl.BlockSpec((bm, k), lambda i: (i, 0)),
        pl.BlockSpec((k, n), lambda i: (0, 0)),
    ]
    args = [x, w]
    if b is not None:
        in_specs.append(pl.BlockSpec((n,), lambda i: (0,)))
        args.append(b)
        body = functools.partial(_mm_bias_kernel, act=act, slope=SLOPE)
    else:
        body = functools.partial(_mm_kernel, act=act, slope=SLOPE)
    return pl.pallas_call(
        body,
        grid=grid,
        in_specs=in_specs,
        out_specs=pl.BlockSpec((bm, n), lambda i: (i, 0)),
        out_shape=jax.ShapeDtypeStruct((m, n), jnp.float32),
    )(*args)


def _readout_kernel(x_ref, wo_ref, wr_ref, o_ref, acc_ref):
    i = pl.program_id(0)

    @pl.when(i == 0)
    def _():
        acc_ref[...] = jnp.zeros_like(acc_ref)

    acc_ref[...] += jnp.dot(
        wo_ref[...].T, x_ref[...], preferred_element_type=jnp.float32
    )

    @pl.when(i == pl.num_programs(0) - 1)
    def _():
        o_ref[...] = jnp.sum(acc_ref[...] * wr_ref[...].T)[None, None]


def _readout(x, w_out_slice, w_ro):
    m, c = x.shape
    bm = _pick_bm(m)
    return pl.pallas_call(
        _readout_kernel,
        grid=(m // bm,),
        in_specs=[
            pl.BlockSpec((bm, c), lambda i: (i, 0)),
            pl.BlockSpec((bm, 1), lambda i: (i, 0)),
            pl.BlockSpec((c, 1), lambda i: (0, 0)),
        ],
        out_specs=pl.BlockSpec((1, 1), lambda i: (0, 0)),
        out_shape=jax.ShapeDtypeStruct((1, 1), jnp.float32),
        scratch_shapes=[pltpu.VMEM((1, c), jnp.float32)],
    )(x, w_out_slice, w_ro)


def _lap_matvec(x, ei, ew, n):
    return jnp.zeros((n, x.shape[-1]), x.dtype).at[ei[1]].add(ew[:, None] * x[ei[0]])


def _laguerre_conv(x, ei, ew, w, act=True):
    n = x.shape[0]
    t1 = x - _lap_matvec(x, ei, ew, n)
    return _mm(jnp.concatenate([x, t1], axis=-1), w, act=act)


def kernel(x_t, edge_index_t, edge_weight_t, x_s, edge_index_s, edge_weight_s,
           edge_index, params):
    p = params
    xt = _mm(x_t, p["W_emb"], b=p["b_emb"], act=True)
    x_t0 = _laguerre_conv(xt, edge_index_t, edge_weight_t, p["W_init_t"])
    x_s0 = _laguerre_conv(x_s, edge_index_s, edge_weight_s, p["W_init_s"])

    deg = jnp.zeros((N,), jnp.float32).at[edge_index.reshape(-1)].add(1.0) + 1e-6
    inv_deg = 1.0 / deg

    for i in range(3):
        ot = _laguerre_conv(x_t0, edge_index_t, edge_weight_t, p["W_hl_t"][i])
        os_ = _laguerre_conv(x_s0, edge_index_s, edge_weight_s, p["W_hl_s"][i])
        x_t0 = jnp.concatenate([x_t0, ot], axis=-1)
        x_s0 = jnp.concatenate([x_s0, os_], axis=-1)
        agg_t = (
            jnp.zeros((N, x_s0.shape[-1]), jnp.float32)
            .at[edge_index[1]].add(x_s0)
            .at[edge_index[0]].add(-x_s0)
        )
        m_t = agg_t * inv_deg[:, None]
        m_s = x_t0[edge_index[1]] - x_t0[edge_index[0]]
        x_t0 = jnp.concatenate([x_t0, _mm(m_t, p["W_msi_t"][i], act=True)], axis=-1)
        x_s0 = jnp.concatenate([x_s0, _mm(m_s, p["W_msi_s"][i], act=True)], axis=-1)

    out_s = _readout(x_s0, p["W_out"][:E], p["W_ro_s"])
    out_t = _readout(x_t0, p["W_out"][E:], p["W_ro_t"])
    return out_s + out_t + p["b_out"][None, :]


# speedup vs baseline: 1.7573x; 1.7573x over previous
import functools

import jax
import jax.numpy as jnp
from jax import lax
from jax.experimental import pallas as pl
from jax.experimental.pallas import tpu as pltpu
from jax.experimental.pallas import tpu_sc as plsc

N = 10000
E = 160000
ES = 320000
SLOPE = 0.1

NC, NS, L = 2, 16, 16
NW = NC * NS
CHUNK = 128

_SC_PARAMS = pltpu.CompilerParams(use_tc_tiling_on_sc=False,
                                  needs_layout_passes=False)
_MESH = plsc.VectorSubcoreMesh(core_axis_name="c", subcore_axis_name="s")


@functools.lru_cache(maxsize=None)
def _make_spmv_small(n_out, c, n_edges):
    nch = n_edges // CHUNK
    rpt = n_out // NS

    @functools.partial(
        pl.kernel, mesh=_MESH, compiler_params=_SC_PARAMS,
        out_type=jax.ShapeDtypeStruct((NC, n_out, c), jnp.float32),
        scratch_types=[
            pltpu.VMEM((CHUNK,), jnp.int32),
            pltpu.VMEM((CHUNK,), jnp.int32),
            pltpu.VMEM((CHUNK,), jnp.float32),
            pltpu.VMEM((CHUNK, c), jnp.float32),
            pltpu.VMEM_SHARED((n_out, c), jnp.float32),
            pltpu.SemaphoreType.DMA,
        ],
    )
    def k(y_hbm, src_hbm, dst_hbm, w_hbm, zeros_hbm, out_hbm,
          src_v, dst_v, w_v, rows_v, acc, sem):
        cid = lax.axis_index("c")
        sid = lax.axis_index("s")
        wid = sid * NC + cid
        pltpu.sync_copy(zeros_hbm.at[pl.ds(sid * rpt, rpt)],
                        acc.at[pl.ds(sid * rpt, rpt)])
        plsc.subcore_barrier()
        max_n = -(-nch // NW)

        def body(kk, carry):
            cc = wid + kk * NW

            @pl.when(cc < nch)
            def _():
                base = cc * CHUNK
                pltpu.sync_copy(src_hbm.at[pl.ds(base, CHUNK)], src_v)
                pltpu.sync_copy(dst_hbm.at[pl.ds(base, CHUNK)], dst_v)
                pltpu.sync_copy(w_hbm.at[pl.ds(base, CHUNK)], w_v)
                pltpu.async_copy(y_hbm.at[src_v], rows_v, sem).wait()
                for j in range(CHUNK // L):
                    w16 = w_v[pl.ds(j * L, L)]
                    for e in range(L):
                        spl = jnp.take(w16, jnp.full((L,), e, jnp.int32))
                        r = j * L + e
                        for t in range(c // L):
                            rows_v[r, pl.ds(t * L, L)] = (
                                rows_v[r, pl.ds(t * L, L)] * spl)
                pltpu.sync_copy(rows_v, acc.at[dst_v], add=True)
            return carry

        lax.fori_loop(0, max_n, body, 0)
        plsc.subcore_barrier()
        pltpu.sync_copy(acc.at[pl.ds(sid * rpt, rpt)],
                        out_hbm.at[cid, pl.ds(sid * rpt, rpt)])

    return k


@functools.lru_cache(maxsize=None)
def _make_spmv_big(c, n_edges):
    nslab = c // 8
    nch = n_edges // CHUNK
    rpt = E // NS

    @functools.partial(
        pl.kernel, mesh=_MESH, compiler_params=_SC_PARAMS,
        out_type=jax.ShapeDtypeStruct((E, c), jnp.float32),
        scratch_types=[
            pltpu.VMEM((CHUNK,), jnp.int32),
            pltpu.VMEM((CHUNK,), jnp.int32),
            pltpu.VMEM((CHUNK,), jnp.int32),
            pltpu.VMEM((CHUNK,), jnp.float32),
            pltpu.VMEM((CHUNK, 8), jnp.float32),
            pltpu.VMEM_SHARED((E, 8), jnp.float32),
            pltpu.SemaphoreType.DMA,
        ],
    )
    def k(yv_hbm, src_hbm, dst_hbm, w_hbm, zeros_hbm, out_hbm,
          src_v, dst_v, gidx_v, w_v, rows_v, acc, sem):
        cid = lax.axis_index("c")
        sid = lax.axis_index("s")
        iot = lax.iota(jnp.int32, L)
        rowb = jnp.right_shift(iot, 3)
        colx = jnp.bitwise_and(iot, 7)
        max_n = -(-nch // NS)
        for s in range(nslab):
            @pl.when(cid == (s % NC))
            def _():
                pltpu.sync_copy(zeros_hbm.at[pl.ds(sid * rpt, rpt)],
                                acc.at[pl.ds(sid * rpt, rpt)])
                plsc.subcore_barrier()

                def body(kk, carry):
                    cc = sid + kk * NS

                    @pl.when(cc < nch)
                    def _():
                        base = cc * CHUNK
                        pltpu.sync_copy(src_hbm.at[pl.ds(base, CHUNK)], src_v)
                        pltpu.sync_copy(dst_hbm.at[pl.ds(base, CHUNK)], dst_v)
                        pltpu.sync_copy(w_hbm.at[pl.ds(base, CHUNK)], w_v)
                        for j in range(CHUNK // L):
                            sv = src_v[pl.ds(j * L, L)]
                            gidx_v[pl.ds(j * L, L)] = sv * nslab + s
                        pltpu.async_copy(yv_hbm.at[gidx_v], rows_v, sem).wait()
                        wsl = [w_v[pl.ds(j * L, L)] for j in range(CHUNK // L)]
                        for g in range(CHUNK // 2):
                            wspl = jnp.take(wsl[g // 8], rowb + ((2 * g) % L))
                            ridx = rowb + (2 * g)
                            v = plsc.load_gather(rows_v, [ridx, colx])
                            plsc.store_scatter(rows_v, [ridx, colx], v * wspl)
                        pltpu.sync_copy(rows_v, acc.at[dst_v], add=True)
                    return carry

                lax.fori_loop(0, max_n, body, 0)
                plsc.subcore_barrier()
                pltpu.sync_copy(acc.at[pl.ds(sid * rpt, rpt)],
                                out_hbm.at[pl.ds(sid * rpt, rpt),
                                           pl.ds(s * 8, 8)])

    return k


@functools.lru_cache(maxsize=None)
def _make_agg(c):
    nch = E // CHUNK
    rpt = N // NS

    @functools.partial(
        pl.kernel, mesh=_MESH, compiler_params=_SC_PARAMS,
        out_type=jax.ShapeDtypeStruct((NC, N, c), jnp.float32),
        scratch_types=[
            pltpu.VMEM((CHUNK,), jnp.int32),
            pltpu.VMEM((CHUNK,), jnp.int32),
            pltpu.VMEM((CHUNK, c), jnp.float32),
            pltpu.VMEM_SHARED((N, c), jnp.float32),
        ],
    )
    def k(z_hbm, src_hbm, dst_hbm, zeros_hbm, out_hbm,
          src_v, dst_v, rows_v, acc):
        cid = lax.axis_index("c")
        sid = lax.axis_index("s")
        wid = sid * NC + cid
        pltpu.sync_copy(zeros_hbm.at[pl.ds(sid * rpt, rpt)],
                        acc.at[pl.ds(sid * rpt, rpt)])
        plsc.subcore_barrier()
        max_n = -(-nch // NW)

        def body(kk, carry):
            cc = wid + kk * NW

            @pl.when(cc < nch)
            def _():
                base = cc * CHUNK
                pltpu.sync_copy(src_hbm.at[pl.ds(base, CHUNK)], src_v)
                pltpu.sync_copy(dst_hbm.at[pl.ds(base, CHUNK)], dst_v)
                pltpu.sync_copy(z_hbm.at[pl.ds(base, CHUNK)], rows_v)
                pltpu.sync_copy(rows_v, acc.at[dst_v], add=True)
                for r in range(CHUNK):
                    for t in range(c // L):
                        rows_v[r, pl.ds(t * L, L)] = (
                            0.0 - rows_v[r, pl.ds(t * L, L)])
                pltpu.sync_copy(rows_v, acc.at[src_v], add=True)
            return carry

        lax.fori_loop(0, max_n, body, 0)
        plsc.subcore_barrier()
        pltpu.sync_copy(acc.at[pl.ds(sid * rpt, rpt)],
                        out_hbm.at[cid, pl.ds(sid * rpt, rpt)])

    return k


@functools.lru_cache(maxsize=None)
def _make_deg():
    nch = (2 * E) // CHUNK
    rpt = N // NS

    @functools.partial(
        pl.kernel, mesh=_MESH, compiler_params=_SC_PARAMS,
        out_type=jax.ShapeDtypeStruct((NC, N, L), jnp.float32),
        scratch_types=[
            pltpu.VMEM((CHUNK,), jnp.int32),
            pltpu.VMEM((CHUNK, L), jnp.float32),
            pltpu.VMEM_SHARED((N, L), jnp.float32),
        ],
    )
    def k(idx_hbm, ones_hbm, zeros_hbm, out_hbm, idx_v, ones_v, acc):
        cid = lax.axis_index("c")
        sid = lax.axis_index("s")
        wid = sid * NC + cid
        pltpu.sync_copy(zeros_hbm.at[pl.ds(sid * rpt, rpt)],
                        acc.at[pl.ds(sid * rpt, rpt)])
        pltpu.sync_copy(ones_hbm, ones_v)
        plsc.subcore_barrier()
        max_n = -(-nch // NW)

        def body(kk, carry):
            cc = wid + kk * NW

            @pl.when(cc < nch)
            def _():
                base = cc * CHUNK
                pltpu.sync_copy(idx_hbm.at[pl.ds(base, CHUNK)], idx_v)
                pltpu.sync_copy(ones_v, acc.at[idx_v], add=True)
            return carry

        lax.fori_loop(0, max_n, body, 0)
        plsc.subcore_barrier()
        pltpu.sync_copy(acc.at[pl.ds(sid * rpt, rpt)],
                        out_hbm.at[cid, pl.ds(sid * rpt, rpt)])

    return k


@functools.lru_cache(maxsize=None)
def _make_gdiff(c):
    nch = E // CHUNK

    @functools.partial(
        pl.kernel, mesh=_MESH, compiler_params=_SC_PARAMS,
        out_type=jax.ShapeDtypeStruct((E, c), jnp.float32),
        scratch_types=[
            pltpu.VMEM((CHUNK,), jnp.int32),
            pltpu.VMEM((CHUNK,), jnp.int32),
            pltpu.VMEM((CHUNK, c), jnp.float32),
            pltpu.VMEM((CHUNK, c), jnp.float32),
            pltpu.SemaphoreType.DMA,
        ],
    )
    def k(y_hbm, src_hbm, dst_hbm, out_hbm, src_v, dst_v, rows_d, rows_s, sem):
        cid = lax.axis_index("c")
        sid = lax.axis_index("s")
        wid = sid * NC + cid
        max_n = -(-nch // NW)

        def body(kk, carry):
            cc = wid + kk * NW

            @pl.when(cc < nch)
            def _():
                base = cc * CHUNK
                pltpu.sync_copy(src_hbm.at[pl.ds(base, CHUNK)], src_v)
                pltpu.sync_copy(dst_hbm.at[pl.ds(base, CHUNK)], dst_v)
                cp1 = pltpu.async_copy(y_hbm.at[dst_v], rows_d, sem)
                cp2 = pltpu.async_copy(y_hbm.at[src_v], rows_s, sem)
                cp1.wait()
                cp2.wait()
                for r in range(CHUNK):
                    for t in range(c // L):
                        rows_d[r, pl.ds(t * L, L)] = (
                            rows_d[r, pl.ds(t * L, L)]
                            - rows_s[r, pl.ds(t * L, L)])
                pltpu.sync_copy(rows_d, out_hbm.at[pl.ds(base, CHUNK)])
            return carry

        lax.fori_loop(0, max_n, body, 0)

    return k


_BM = 2000


def _tc_linear(x, w, *, bias=None, act=True):
    m, kdim = x.shape
    n = w.shape[1]
    in_specs = [
        pl.BlockSpec((_BM, kdim), lambda i: (i, 0)),
        pl.BlockSpec((kdim, n), lambda i: (0, 0)),
    ]
    args = [x, w]
    if bias is not None:
        in_specs.append(pl.BlockSpec((n,), lambda i: (0,)))
        args.append(bias)

    def body(*refs):
        o_ref = refs[-1]
        acc = jnp.dot(refs[0][...], refs[1][...],
                      preferred_element_type=jnp.float32)
        if bias is not None:
            acc = acc + refs[2][...][None, :]
        if act:
            acc = jnp.maximum(acc, SLOPE * acc)
        o_ref[...] = acc

    return pl.pallas_call(
        body,
        grid=(m // _BM,),
        in_specs=in_specs,
        out_specs=pl.BlockSpec((_BM, n), lambda i: (i, 0)),
        out_shape=jax.ShapeDtypeStruct((m, n), jnp.float32),
    )(*args)


def _tc_conv(x, w, p0, p1=None, act=True):
    m, kdim = x.shape
    n = w.shape[1]
    in_specs = [
        pl.BlockSpec((_BM, kdim), lambda i: (i, 0)),
        pl.BlockSpec((2 * kdim, n), lambda i: (0, 0)),
        pl.BlockSpec((_BM, kdim), lambda i: (i, 0)),
    ]
    args = [x, w, p0]
    if p1 is not None:
        in_specs.append(pl.BlockSpec((_BM, kdim), lambda i: (i, 0)))
        args.append(p1)

    def body(*refs):
        o_ref = refs[-1]
        xv = refs[0][...]
        t1 = xv - refs[2][...]
        if p1 is not None:
            t1 = t1 - refs[3][...]
        cc = jnp.concatenate([xv, t1], axis=-1)
        acc = jnp.dot(cc, refs[1][...], preferred_element_type=jnp.float32)
        if act:
            acc = jnp.maximum(acc, SLOPE * acc)
        o_ref[...] = acc

    return pl.pallas_call(
        body,
        grid=(m // _BM,),
        in_specs=in_specs,
        out_specs=pl.BlockSpec((_BM, n), lambda i: (i, 0)),
        out_shape=jax.ShapeDtypeStruct((m, n), jnp.float32),
    )(*args)


def _tc_linear2(xa, xb, w, *, act=True):
    m = xa.shape[0]
    n = w.shape[1]

    def body(xa_r, xb_r, w_r, o_ref):
        cc = jnp.concatenate([xa_r[...], xb_r[...]], axis=-1)
        acc = jnp.dot(cc, w_r[...], preferred_element_type=jnp.float32)
        if act:
            acc = jnp.maximum(acc, SLOPE * acc)
        o_ref[...] = acc

    return pl.pallas_call(
        body,
        grid=(m // _BM,),
        in_specs=[
            pl.BlockSpec((_BM, xa.shape[1]), lambda i: (i, 0)),
            pl.BlockSpec((_BM, xb.shape[1]), lambda i: (i, 0)),
            pl.BlockSpec(w.shape, lambda i: (0, 0)),
        ],
        out_specs=pl.BlockSpec((_BM, n), lambda i: (i, 0)),
        out_shape=jax.ShapeDtypeStruct((m, n), jnp.float32),
    )(xa, xb, w)


def _mdiv(q0, q1, d0, d1):
    c = q0.shape[1]

    def body(q0r, q1r, d0r, d1r, o_ref):
        deg = d0r[...][:, :1] + d1r[...][:, :1] + 1e-6
        o_ref[...] = (q0r[...] + q1r[...]) / deg

    return pl.pallas_call(
        body,
        grid=(N // _BM,),
        in_specs=[
            pl.BlockSpec((_BM, c), lambda i: (i, 0)),
            pl.BlockSpec((_BM, c), lambda i: (i, 0)),
            pl.BlockSpec((_BM, L), lambda i: (i, 0)),
            pl.BlockSpec((_BM, L), lambda i: (i, 0)),
        ],
        out_specs=pl.BlockSpec((_BM, c), lambda i: (i, 0)),
        out_shape=jax.ShapeDtypeStruct((N, c), jnp.float32),
    )(q0, q1, d0, d1)


def _tc_rowdot(x, w):
    m, kdim = x.shape

    def body(x_ref, w_ref, o_ref):
        o_ref[...] = jnp.sum(x_ref[...] * w_ref[...][:, 0][None, :],
                             axis=1, keepdims=True)

    return pl.pallas_call(
        body,
        grid=(m // _BM,),
        in_specs=[
            pl.BlockSpec((_BM, kdim), lambda i: (i, 0)),
            pl.BlockSpec((kdim, 1), lambda i: (0, 0)),
        ],
        out_specs=pl.BlockSpec((_BM, 1), lambda i: (i, 0)),
        out_shape=jax.ShapeDtypeStruct((m, 1), jnp.float32),
    )(x, w)


def _dot1(r, wo):
    m = r.shape[0]

    def body(r_ref, wo_ref, o_ref):
        i = pl.program_id(0)

        @pl.when(i == 0)
        def _():
            o_ref[...] = jnp.zeros_like(o_ref)

        o_ref[...] += jnp.sum(r_ref[...] * wo_ref[...])[None, None]

    return pl.pallas_call(
        body,
        grid=(m // _BM,),
        in_specs=[
            pl.BlockSpec((_BM, 1), lambda i: (i, 0)),
            pl.BlockSpec((_BM, 1), lambda i: (i, 0)),
        ],
        out_specs=pl.BlockSpec((1, 1), lambda i: (0, 0)),
        out_shape=jax.ShapeDtypeStruct((1, 1), jnp.float32),
    )(r, wo)


def kernel(x_t, edge_index_t, edge_weight_t, x_s, edge_index_s, edge_weight_s,
           edge_index, params):
    p = params
    src_t, dst_t = edge_index_t[0], edge_index_t[1]
    src_s, dst_s = edge_index_s[0], edge_index_s[1]
    bsrc, bdst = edge_index[0], edge_index[1]

    zeros_n = {c: jnp.zeros((N, c), jnp.float32)
               for c in (16, 32, 48, 64, 80, 112, 176)}
    zeros_e8 = jnp.zeros((E, 8), jnp.float32)
    ones_c = jnp.ones((CHUNK, L), jnp.float32)

    xt = _tc_linear(x_t, p["W_emb"], bias=p["b_emb"], act=True)

    lap = _make_spmv_small(N, 64, E)(xt, src_t, dst_t, edge_weight_t,
                                     zeros_n[64])
    x_t0 = _tc_conv(xt, p["W_init_t"], lap[0], lap[1], act=True)

    xs8 = jnp.tile(x_s, (1, 8))
    ls = _make_spmv_big(8, ES)(xs8, src_s, dst_s, edge_weight_s,
                               zeros_e8)[:, :1]
    x_s0 = _tc_conv(x_s, p["W_init_s"], ls, act=True)

    deg = _make_deg()(edge_index.reshape(-1), ones_c, zeros_n[16])

    for i in range(3):
        cin = x_t0.shape[1]
        cout = (16, 32, 64)[i]

        lap_t = _make_spmv_small(N, cin, E)(
            x_t0, src_t, dst_t, edge_weight_t, zeros_n[cin])
        ot = _tc_conv(x_t0, p["W_hl_t"][i], lap_t[0], lap_t[1], act=True)

        lap_s = _make_spmv_big(cin, ES)(
            x_s0.reshape(E * (cin // 8), 8), src_s, dst_s, edge_weight_s,
            zeros_e8)
        os_ = _tc_conv(x_s0, p["W_hl_s"][i], lap_s, act=True)

        q_a = _make_agg(cin)(x_s0, bsrc, bdst, zeros_n[cin])
        q_b = _make_agg(cout)(os_, bsrc, bdst, zeros_n[cout])

        x_t0 = jnp.concatenate([x_t0, ot], axis=-1)
        x_s0 = jnp.concatenate([x_s0, os_], axis=-1)
        cmid = x_t0.shape[1]

        m_a = _mdiv(q_a[0], q_a[1], deg[0], deg[1])
        m_b = _mdiv(q_b[0], q_b[1], deg[0], deg[1])
        mt = _tc_linear2(m_a, m_b, p["W_msi_t"][i], act=True)

        m_s = _make_gdiff(cmid)(x_t0, bsrc, bdst)
        ms = _tc_linear(m_s, p["W_msi_s"][i], act=True)

        x_t0 = jnp.concatenate([x_t0, mt], axis=-1)
        x_s0 = jnp.concatenate([x_s0, ms], axis=-1)

    r_s = _tc_linear(x_s0, p["W_ro_s"], act=False)
    r_t = _tc_linear(x_t0, p["W_ro_t"], act=False)
    out_s = _dot1(r_s, p["W_out"][:E])
    out_t = _dot1(r_t, p["W_out"][E:])
    return out_s + out_t + p["b_out"][None, :]


# speedup vs baseline: 1.9016x; 1.0821x over previous
import functools

import jax
import jax.numpy as jnp
from jax import lax
from jax.experimental import pallas as pl
from jax.experimental.pallas import tpu as pltpu
from jax.experimental.pallas import tpu_sc as plsc

N = 10000
E = 160000
ES = 320000
SLOPE = 0.1

NC, NS, L = 2, 16, 16
NW = NC * NS
CHUNK = 128

_SC_PARAMS = pltpu.CompilerParams(use_tc_tiling_on_sc=False,
                                  needs_layout_passes=False)
_MESH = plsc.VectorSubcoreMesh(core_axis_name="c", subcore_axis_name="s")


@functools.lru_cache(maxsize=None)
def _make_spmv_small(n_out, c, n_edges):
    nch = n_edges // CHUNK
    rpt = n_out // NS

    @functools.partial(
        pl.kernel, mesh=_MESH, compiler_params=_SC_PARAMS,
        out_type=jax.ShapeDtypeStruct((NC, n_out, c), jnp.float32),
        scratch_types=[
            pltpu.VMEM((CHUNK,), jnp.int32),
            pltpu.VMEM((CHUNK,), jnp.int32),
            pltpu.VMEM((CHUNK,), jnp.float32),
            pltpu.VMEM((CHUNK, c), jnp.float32),
            pltpu.VMEM_SHARED((n_out, c), jnp.float32),
            pltpu.SemaphoreType.DMA,
        ],
    )
    def k(y_hbm, src_hbm, dst_hbm, w_hbm, zeros_hbm, out_hbm,
          src_v, dst_v, w_v, rows_v, acc, sem):
        cid = lax.axis_index("c")
        sid = lax.axis_index("s")
        wid = sid * NC + cid
        pltpu.sync_copy(zeros_hbm.at[pl.ds(sid * rpt, rpt)],
                        acc.at[pl.ds(sid * rpt, rpt)])
        plsc.subcore_barrier()
        max_n = -(-nch // NW)

        def body(kk, carry):
            cc = wid + kk * NW

            @pl.when(cc < nch)
            def _():
                base = cc * CHUNK
                pltpu.sync_copy(src_hbm.at[pl.ds(base, CHUNK)], src_v)
                pltpu.sync_copy(dst_hbm.at[pl.ds(base, CHUNK)], dst_v)
                pltpu.sync_copy(w_hbm.at[pl.ds(base, CHUNK)], w_v)
                pltpu.async_copy(y_hbm.at[src_v], rows_v, sem).wait()
                for j in range(CHUNK // L):
                    w16 = w_v[pl.ds(j * L, L)]
                    for e in range(L):
                        spl = jnp.take(w16, jnp.full((L,), e, jnp.int32))
                        r = j * L + e
                        for t in range(c // L):
                            rows_v[r, pl.ds(t * L, L)] = (
                                rows_v[r, pl.ds(t * L, L)] * spl)
                pltpu.sync_copy(rows_v, acc.at[dst_v], add=True)
            return carry

        lax.fori_loop(0, max_n, body, 0)
        plsc.subcore_barrier()
        pltpu.sync_copy(acc.at[pl.ds(sid * rpt, rpt)],
                        out_hbm.at[cid, pl.ds(sid * rpt, rpt)])

    return k


@functools.lru_cache(maxsize=None)
def _make_spmv_big(c, n_edges):
    nslab = c // 8
    nch = n_edges // CHUNK
    rpt = E // NS

    @functools.partial(
        pl.kernel, mesh=_MESH, compiler_params=_SC_PARAMS,
        out_type=jax.ShapeDtypeStruct((E, c), jnp.float32),
        scratch_types=[
            pltpu.VMEM((CHUNK,), jnp.int32),
            pltpu.VMEM((CHUNK,), jnp.int32),
            pltpu.VMEM((CHUNK,), jnp.int32),
            pltpu.VMEM((CHUNK,), jnp.float32),
            pltpu.VMEM((CHUNK, 8), jnp.float32),
            pltpu.VMEM_SHARED((E, 8), jnp.float32),
            pltpu.SemaphoreType.DMA,
        ],
    )
    def k(yv_hbm, src_hbm, dst_hbm, w_hbm, zeros_hbm, out_hbm,
          src_v, dst_v, gidx_v, w_v, rows_v, acc, sem):
        cid = lax.axis_index("c")
        sid = lax.axis_index("s")
        iot = lax.iota(jnp.int32, L)
        rowb = jnp.right_shift(iot, 3)
        colx = jnp.bitwise_and(iot, 7)
        max_n = -(-nch // NS)
        for s in range(nslab):
            @pl.when(cid == (s % NC))
            def _():
                pltpu.sync_copy(zeros_hbm.at[pl.ds(sid * rpt, rpt)],
                                acc.at[pl.ds(sid * rpt, rpt)])
                plsc.subcore_barrier()

                def body(kk, carry):
                    cc = sid + kk * NS

                    @pl.when(cc < nch)
                    def _():
                        base = cc * CHUNK
                        pltpu.sync_copy(src_hbm.at[pl.ds(base, CHUNK)], src_v)
                        pltpu.sync_copy(dst_hbm.at[pl.ds(base, CHUNK)], dst_v)
                        pltpu.sync_copy(w_hbm.at[pl.ds(base, CHUNK)], w_v)
                        for j in range(CHUNK // L):
                            sv = src_v[pl.ds(j * L, L)]
                            gidx_v[pl.ds(j * L, L)] = sv * nslab + s
                        pltpu.async_copy(yv_hbm.at[gidx_v], rows_v, sem).wait()
                        wsl = [w_v[pl.ds(j * L, L)] for j in range(CHUNK // L)]
                        for g in range(CHUNK // 2):
                            wspl = jnp.take(wsl[g // 8], rowb + ((2 * g) % L))
                            ridx = rowb + (2 * g)
                            v = plsc.load_gather(rows_v, [ridx, colx])
                            plsc.store_scatter(rows_v, [ridx, colx], v * wspl)
                        pltpu.sync_copy(rows_v, acc.at[dst_v], add=True)
                    return carry

                lax.fori_loop(0, max_n, body, 0)
                plsc.subcore_barrier()
                pltpu.sync_copy(acc.at[pl.ds(sid * rpt, rpt)],
                                out_hbm.at[pl.ds(sid * rpt, rpt),
                                           pl.ds(s * 8, 8)])

    return k


@functools.lru_cache(maxsize=None)
def _make_agg(c):
    nch = E // CHUNK
    rpt = N // NS

    @functools.partial(
        pl.kernel, mesh=_MESH, compiler_params=_SC_PARAMS,
        out_type=jax.ShapeDtypeStruct((NC, N, c), jnp.float32),
        scratch_types=[
            pltpu.VMEM((CHUNK,), jnp.int32),
            pltpu.VMEM((CHUNK,), jnp.int32),
            pltpu.VMEM((CHUNK, c), jnp.float32),
            pltpu.VMEM_SHARED((N, c), jnp.float32),
        ],
    )
    def k(z_hbm, src_hbm, dst_hbm, zeros_hbm, out_hbm,
          src_v, dst_v, rows_v, acc):
        cid = lax.axis_index("c")
        sid = lax.axis_index("s")
        wid = sid * NC + cid
        pltpu.sync_copy(zeros_hbm.at[pl.ds(sid * rpt, rpt)],
                        acc.at[pl.ds(sid * rpt, rpt)])
        plsc.subcore_barrier()
        max_n = -(-nch // NW)

        def body(kk, carry):
            cc = wid + kk * NW

            @pl.when(cc < nch)
            def _():
                base = cc * CHUNK
                pltpu.sync_copy(src_hbm.at[pl.ds(base, CHUNK)], src_v)
                pltpu.sync_copy(dst_hbm.at[pl.ds(base, CHUNK)], dst_v)
                pltpu.sync_copy(z_hbm.at[pl.ds(base, CHUNK)], rows_v)
                pltpu.sync_copy(rows_v, acc.at[dst_v], add=True)
                for r in range(CHUNK):
                    for t in range(c // L):
                        rows_v[r, pl.ds(t * L, L)] = (
                            0.0 - rows_v[r, pl.ds(t * L, L)])
                pltpu.sync_copy(rows_v, acc.at[src_v], add=True)
            return carry

        lax.fori_loop(0, max_n, body, 0)
        plsc.subcore_barrier()
        pltpu.sync_copy(acc.at[pl.ds(sid * rpt, rpt)],
                        out_hbm.at[cid, pl.ds(sid * rpt, rpt)])

    return k


@functools.lru_cache(maxsize=None)
def _make_deg():
    nch = (2 * E) // CHUNK
    rpt = N // NS

    @functools.partial(
        pl.kernel, mesh=_MESH, compiler_params=_SC_PARAMS,
        out_type=jax.ShapeDtypeStruct((NC, N, L), jnp.float32),
        scratch_types=[
            pltpu.VMEM((CHUNK,), jnp.int32),
            pltpu.VMEM((CHUNK, L), jnp.float32),
            pltpu.VMEM_SHARED((N, L), jnp.float32),
        ],
    )
    def k(idx_hbm, ones_hbm, zeros_hbm, out_hbm, idx_v, ones_v, acc):
        cid = lax.axis_index("c")
        sid = lax.axis_index("s")
        wid = sid * NC + cid
        pltpu.sync_copy(zeros_hbm.at[pl.ds(sid * rpt, rpt)],
                        acc.at[pl.ds(sid * rpt, rpt)])
        pltpu.sync_copy(ones_hbm, ones_v)
        plsc.subcore_barrier()
        max_n = -(-nch // NW)

        def body(kk, carry):
            cc = wid + kk * NW

            @pl.when(cc < nch)
            def _():
                base = cc * CHUNK
                pltpu.sync_copy(idx_hbm.at[pl.ds(base, CHUNK)], idx_v)
                pltpu.sync_copy(ones_v, acc.at[idx_v], add=True)
            return carry

        lax.fori_loop(0, max_n, body, 0)
        plsc.subcore_barrier()
        pltpu.sync_copy(acc.at[pl.ds(sid * rpt, rpt)],
                        out_hbm.at[cid, pl.ds(sid * rpt, rpt)])

    return k


@functools.lru_cache(maxsize=None)
def _make_gdiff(c, act=False):
    nch = E // CHUNK

    @functools.partial(
        pl.kernel, mesh=_MESH, compiler_params=_SC_PARAMS,
        out_type=jax.ShapeDtypeStruct((E, c), jnp.float32),
        scratch_types=[
            pltpu.VMEM((CHUNK,), jnp.int32),
            pltpu.VMEM((CHUNK,), jnp.int32),
            pltpu.VMEM((CHUNK, c), jnp.float32),
            pltpu.VMEM((CHUNK, c), jnp.float32),
            pltpu.SemaphoreType.DMA,
        ],
    )
    def k(y_hbm, src_hbm, dst_hbm, out_hbm, src_v, dst_v, rows_d, rows_s, sem):
        cid = lax.axis_index("c")
        sid = lax.axis_index("s")
        wid = sid * NC + cid
        max_n = -(-nch // NW)

        def body(kk, carry):
            cc = wid + kk * NW

            @pl.when(cc < nch)
            def _():
                base = cc * CHUNK
                pltpu.sync_copy(src_hbm.at[pl.ds(base, CHUNK)], src_v)
                pltpu.sync_copy(dst_hbm.at[pl.ds(base, CHUNK)], dst_v)
                cp1 = pltpu.async_copy(y_hbm.at[dst_v], rows_d, sem)
                cp2 = pltpu.async_copy(y_hbm.at[src_v], rows_s, sem)
                cp1.wait()
                cp2.wait()
                for r in range(CHUNK):
                    for t in range(c // L):
                        d = (rows_d[r, pl.ds(t * L, L)]
                             - rows_s[r, pl.ds(t * L, L)])
                        if act:
                            d = jnp.maximum(d, SLOPE * d)
                        rows_d[r, pl.ds(t * L, L)] = d
                pltpu.sync_copy(rows_d, out_hbm.at[pl.ds(base, CHUNK)])
            return carry

        lax.fori_loop(0, max_n, body, 0)

    return k


_BM = 2000


def _tc_linear(x, w, *, bias=None, act=True):
    m, kdim = x.shape
    n = w.shape[1]
    in_specs = [
        pl.BlockSpec((_BM, kdim), lambda i: (i, 0)),
        pl.BlockSpec((kdim, n), lambda i: (0, 0)),
    ]
    args = [x, w]
    if bias is not None:
        in_specs.append(pl.BlockSpec((n,), lambda i: (0,)))
        args.append(bias)

    def body(*refs):
        o_ref = refs[-1]
        acc = jnp.dot(refs[0][...], refs[1][...],
                      preferred_element_type=jnp.float32)
        if bias is not None:
            acc = acc + refs[2][...][None, :]
        if act:
            acc = jnp.maximum(acc, SLOPE * acc)
        o_ref[...] = acc

    return pl.pallas_call(
        body,
        grid=(m // _BM,),
        in_specs=in_specs,
        out_specs=pl.BlockSpec((_BM, n), lambda i: (i, 0)),
        out_shape=jax.ShapeDtypeStruct((m, n), jnp.float32),
    )(*args)


def _tc_conv(x, w, p0, p1=None, act=True):
    m, kdim = x.shape
    n = w.shape[1]
    in_specs = [
        pl.BlockSpec((_BM, kdim), lambda i: (i, 0)),
        pl.BlockSpec((2 * kdim, n), lambda i: (0, 0)),
        pl.BlockSpec((_BM, kdim), lambda i: (i, 0)),
    ]
    args = [x, w, p0]
    if p1 is not None:
        in_specs.append(pl.BlockSpec((_BM, kdim), lambda i: (i, 0)))
        args.append(p1)

    def body(*refs):
        o_ref = refs[-1]
        xv = refs[0][...]
        t1 = xv - refs[2][...]
        if p1 is not None:
            t1 = t1 - refs[3][...]
        cc = jnp.concatenate([xv, t1], axis=-1)
        acc = jnp.dot(cc, refs[1][...], preferred_element_type=jnp.float32)
        if act:
            acc = jnp.maximum(acc, SLOPE * acc)
        o_ref[...] = acc

    return pl.pallas_call(
        body,
        grid=(m // _BM,),
        in_specs=in_specs,
        out_specs=pl.BlockSpec((_BM, n), lambda i: (i, 0)),
        out_shape=jax.ShapeDtypeStruct((m, n), jnp.float32),
    )(*args)


def _tc_linear2(xa, xb, w, *, act=True):
    m = xa.shape[0]
    n = w.shape[1]

    def body(xa_r, xb_r, w_r, o_ref):
        cc = jnp.concatenate([xa_r[...], xb_r[...]], axis=-1)
        acc = jnp.dot(cc, w_r[...], preferred_element_type=jnp.float32)
        if act:
            acc = jnp.maximum(acc, SLOPE * acc)
        o_ref[...] = acc

    return pl.pallas_call(
        body,
        grid=(m // _BM,),
        in_specs=[
            pl.BlockSpec((_BM, xa.shape[1]), lambda i: (i, 0)),
            pl.BlockSpec((_BM, xb.shape[1]), lambda i: (i, 0)),
            pl.BlockSpec(w.shape, lambda i: (0, 0)),
        ],
        out_specs=pl.BlockSpec((_BM, n), lambda i: (i, 0)),
        out_shape=jax.ShapeDtypeStruct((m, n), jnp.float32),
    )(xa, xb, w)


def _mdiv(q0, q1, d0, d1):
    c = q0.shape[1]

    def body(q0r, q1r, d0r, d1r, o_ref):
        deg = d0r[...][:, :1] + d1r[...][:, :1] + 1e-6
        o_ref[...] = (q0r[...] + q1r[...]) / deg

    return pl.pallas_call(
        body,
        grid=(N // _BM,),
        in_specs=[
            pl.BlockSpec((_BM, c), lambda i: (i, 0)),
            pl.BlockSpec((_BM, c), lambda i: (i, 0)),
            pl.BlockSpec((_BM, L), lambda i: (i, 0)),
            pl.BlockSpec((_BM, L), lambda i: (i, 0)),
        ],
        out_specs=pl.BlockSpec((_BM, c), lambda i: (i, 0)),
        out_shape=jax.ShapeDtypeStruct((N, c), jnp.float32),
    )(q0, q1, d0, d1)


def _tc_rowdot(x, w):
    m, kdim = x.shape

    def body(x_ref, w_ref, o_ref):
        o_ref[...] = jnp.sum(x_ref[...] * w_ref[...][:, 0][None, :],
                             axis=1, keepdims=True)

    return pl.pallas_call(
        body,
        grid=(m // _BM,),
        in_specs=[
            pl.BlockSpec((_BM, kdim), lambda i: (i, 0)),
            pl.BlockSpec((kdim, 1), lambda i: (0, 0)),
        ],
        out_specs=pl.BlockSpec((_BM, 1), lambda i: (i, 0)),
        out_shape=jax.ShapeDtypeStruct((m, 1), jnp.float32),
    )(x, w)


def _dot1(r, wo):
    m = r.shape[0]

    def body(r_ref, wo_ref, o_ref):
        i = pl.program_id(0)

        @pl.when(i == 0)
        def _():
            o_ref[...] = jnp.zeros_like(o_ref)

        o_ref[...] += jnp.sum(r_ref[...] * wo_ref[...])[None, None]

    return pl.pallas_call(
        body,
        grid=(m // _BM,),
        in_specs=[
            pl.BlockSpec((_BM, 1), lambda i: (i, 0)),
            pl.BlockSpec((_BM, 1), lambda i: (i, 0)),
        ],
        out_specs=pl.BlockSpec((1, 1), lambda i: (0, 0)),
        out_shape=jax.ShapeDtypeStruct((1, 1), jnp.float32),
    )(r, wo)


def kernel(x_t, edge_index_t, edge_weight_t, x_s, edge_index_s, edge_weight_s,
           edge_index, params):
    p = params
    src_t, dst_t = edge_index_t[0], edge_index_t[1]
    src_s, dst_s = edge_index_s[0], edge_index_s[1]
    bsrc, bdst = edge_index[0], edge_index[1]

    zeros_n = {c: jnp.zeros((N, c), jnp.float32)
               for c in (16, 32, 48, 64, 80, 112, 176)}
    zeros_e8 = jnp.zeros((E, 8), jnp.float32)
    ones_c = jnp.ones((CHUNK, L), jnp.float32)

    xt = _tc_linear(x_t, p["W_emb"], bias=p["b_emb"], act=True)

    lap = _make_spmv_small(N, 64, E)(xt, src_t, dst_t, edge_weight_t,
                                     zeros_n[64])
    x_t0 = _tc_conv(xt, p["W_init_t"], lap[0], lap[1], act=True)

    xs8 = jnp.tile(x_s, (1, 8))
    ls = _make_spmv_big(8, ES)(xs8, src_s, dst_s, edge_weight_s,
                               zeros_e8)[:, :1]
    x_s0 = _tc_conv(x_s, p["W_init_s"], ls, act=True)

    deg = _make_deg()(edge_index.reshape(-1), ones_c, zeros_n[16])

    for i in range(3):
        cin = x_t0.shape[1]
        cout = (16, 32, 64)[i]

        lap_t = _make_spmv_small(N, cin, E)(
            x_t0, src_t, dst_t, edge_weight_t, zeros_n[cin])
        ot = _tc_conv(x_t0, p["W_hl_t"][i], lap_t[0], lap_t[1], act=True)

        lap_s = _make_spmv_big(cin, ES)(
            x_s0.reshape(E * (cin // 8), 8), src_s, dst_s, edge_weight_s,
            zeros_e8)
        os_ = _tc_conv(x_s0, p["W_hl_s"][i], lap_s, act=True)

        q_a = _make_agg(cin)(x_s0, bsrc, bdst, zeros_n[cin])
        q_b = _make_agg(cout)(os_, bsrc, bdst, zeros_n[cout])

        x_t0 = jnp.concatenate([x_t0, ot], axis=-1)
        x_s0 = jnp.concatenate([x_s0, os_], axis=-1)
        cmid = x_t0.shape[1]

        m_a = _mdiv(q_a[0], q_a[1], deg[0], deg[1])
        m_b = _mdiv(q_b[0], q_b[1], deg[0], deg[1])
        mt = _tc_linear2(m_a, m_b, p["W_msi_t"][i], act=True)

        y_ms = _tc_linear(x_t0, p["W_msi_s"][i], act=False)
        ms = _make_gdiff(cout, True)(y_ms, bsrc, bdst)

        x_t0 = jnp.concatenate([x_t0, mt], axis=-1)
        x_s0 = jnp.concatenate([x_s0, ms], axis=-1)

    r_s = _tc_linear(x_s0, p["W_ro_s"], act=False)
    r_t = _tc_linear(x_t0, p["W_ro_t"], act=False)
    out_s = _dot1(r_s, p["W_out"][:E])
    out_t = _dot1(r_t, p["W_out"][E:])
    return out_s + out_t + p["b_out"][None, :]
